# baseline (speedup 1.0000x reference)
import functools

import jax
import jax.numpy as jnp
from jax import lax
from jax.experimental import pallas as pl
from jax.experimental.pallas import tpu as pltpu
from jax.experimental.pallas import tpu_sc as plsc

B = 4
S = 4096
H = 1024
EPS = 1e-12
L = 16
NC, NS = 2, 16
NW = NC * NS
POS_PER_TILE = S // NW
PB = 16
NSB = POS_PER_TILE // PB
NCH = H // L


def _rsqrt(x):
    i = lax.bitcast_convert_type(x, jnp.int32)
    i = jnp.int32(0x5F3759DF) - lax.shift_right_logical(i, 1)
    y = lax.bitcast_convert_type(i, jnp.float32)
    for _ in range(4):
        y = y * (jnp.float32(1.5) - jnp.float32(0.5) * x * y * y)
    return y


_GATHER_DNUMS = lax.GatherDimensionNumbers(
    offset_dims=(), collapsed_slice_dims=(0,), start_index_map=(0,))


def _shuffle(v, perm):
    return lax.gather(v, perm[:, None], _GATHER_DNUMS, slice_sizes=(1,),
                      mode=lax.GatherScatterMode.PROMISE_IN_BOUNDS)


def _lane_sum(v):
    iota = lax.iota(jnp.int32, L)
    for k in (8, 4, 2, 1):
        v = v + _shuffle(v, jnp.bitwise_xor(iota, jnp.int32(k)))
    return v


def kernel(input_ids, word_emb, pos_emb, type_emb, gamma, beta):
    mesh = plsc.VectorSubcoreMesh(core_axis_name="c", subcore_axis_name="s")

    @functools.partial(
        pl.kernel,
        mesh=mesh,
        out_type=jax.ShapeDtypeStruct((B, S, H), jnp.float32),
        scratch_types=[
            pltpu.VMEM((PB,), jnp.int32),
            pltpu.VMEM((PB, H), jnp.float32),
            pltpu.VMEM((PB, H), jnp.float32),
            pltpu.VMEM((1, H), jnp.float32),
            pltpu.VMEM((H,), jnp.float32),
            pltpu.VMEM((H,), jnp.float32),
            pltpu.SemaphoreType.DMA,
        ],
    )
    def run(ids_hbm, word_hbm, pos_hbm, type_hbm, gamma_hbm, beta_hbm,
            out_hbm, idx_v, bias_v, rows_v, type_v, gamma_v, beta_v, sem):
        wid = lax.axis_index("s") * NC + lax.axis_index("c")
        pltpu.sync_copy(type_hbm.at[pl.ds(0, 1)], type_v)
        pltpu.sync_copy(gamma_hbm, gamma_v)
        pltpu.sync_copy(beta_hbm, beta_v)

        def sub_block(sb, _):
            p0 = wid * POS_PER_TILE + sb * PB
            pltpu.sync_copy(pos_hbm.at[pl.ds(p0, PB)], bias_v)

            def add_type(p, _):
                def add_chunk(j, _):
                    sl = pl.ds(j * L, L)
                    bias_v[p, sl] = bias_v[p, sl] + type_v[0, sl]
                    return 0
                return lax.fori_loop(0, NCH, add_chunk, 0)
            lax.fori_loop(0, PB, add_type, 0)

            for b in range(B):
                pltpu.sync_copy(ids_hbm.at[b, pl.ds(p0, PB)], idx_v)
                pltpu.async_copy(word_hbm.at[idx_v], rows_v, sem).wait()

                def token(t, _):
                    def stats(j, carry):
                        s, sq = carry
                        sl = pl.ds(j * L, L)
                        x = rows_v[t, sl] + bias_v[t, sl]
                        return s + x, sq + x * x

                    zero = jnp.zeros((L,), jnp.float32)
                    s, sq = lax.fori_loop(0, NCH, stats, (zero, zero))
                    mean = _lane_sum(s) * jnp.float32(1.0 / H)
                    ex2 = _lane_sum(sq) * jnp.float32(1.0 / H)
                    var = ex2 - mean * mean
                    rstd = _rsqrt(var + jnp.float32(EPS))
                    shift = -mean * rstd

                    def norm(j, _):
                        sl = pl.ds(j * L, L)
                        x = rows_v[t, sl] + bias_v[t, sl]
                        y = x * rstd + shift
                        rows_v[t, sl] = y * gamma_v[sl] + beta_v[sl]
                        return 0
                    return lax.fori_loop(0, NCH, norm, 0)
                lax.fori_loop(0, PB, token, 0)
                pltpu.sync_copy(rows_v, out_hbm.at[b, pl.ds(p0, PB)])
            return 0
        lax.fori_loop(0, NSB, sub_block, 0)

    return run(input_ids, word_emb, pos_emb, type_emb, gamma, beta)


# speedup vs baseline: 1.4239x; 1.4239x over previous
import functools

import jax
import jax.numpy as jnp
from jax import lax
from jax.experimental import pallas as pl
from jax.experimental.pallas import tpu as pltpu
from jax.experimental.pallas import tpu_sc as plsc

B = 4
S = 4096
H = 1024
EPS = 1e-12
L = 16
NC, NS = 2, 16
NW = NC * NS
PPT = S // NW
PB = 8
NSB = PPT // PB
NCH = H // L
U = 8


def _rsqrt(x):
    i = lax.bitcast_convert_type(x, jnp.int32)
    i = jnp.int32(0x5F3759DF) - lax.shift_right_logical(i, 1)
    y = lax.bitcast_convert_type(i, jnp.float32)
    for _ in range(4):
        y = y * (jnp.float32(1.5) - jnp.float32(0.5) * x * y * y)
    return y


_GATHER_DNUMS = lax.GatherDimensionNumbers(
    offset_dims=(), collapsed_slice_dims=(0,), start_index_map=(0,))


def _shuffle(v, perm):
    return lax.gather(v, perm[:, None], _GATHER_DNUMS, slice_sizes=(1,),
                      mode=lax.GatherScatterMode.PROMISE_IN_BOUNDS)


def _lane_sum(v):
    iota = lax.iota(jnp.int32, L)
    for k in (8, 4, 2, 1):
        v = v + _shuffle(v, jnp.bitwise_xor(iota, jnp.int32(k)))
    return v


def kernel(input_ids, word_emb, pos_emb, type_emb, gamma, beta):
    mesh = plsc.VectorSubcoreMesh(core_axis_name="c", subcore_axis_name="s")

    @functools.partial(
        pl.kernel,
        mesh=mesh,
        out_type=jax.ShapeDtypeStruct((B, S, H), jnp.float32),
        scratch_types=[
            pltpu.VMEM((B, PPT), jnp.int32),
            pltpu.VMEM((2, PB, H), jnp.float32),
            pltpu.VMEM((2, B, PB, H), jnp.float32),
            pltpu.VMEM((B, PB, H), jnp.float32),
            pltpu.VMEM((2, B, PB, L), jnp.float32),
            pltpu.VMEM((1, H), jnp.float32),
            pltpu.VMEM((H,), jnp.float32),
            pltpu.VMEM((H,), jnp.float32),
            pltpu.SemaphoreType.DMA,
            pltpu.SemaphoreType.DMA,
            pltpu.SemaphoreType.DMA,
            pltpu.SemaphoreType.DMA,
            pltpu.SemaphoreType.DMA,
        ],
    )
    def run(ids_hbm, word_hbm, pos_hbm, type_hbm, gamma_hbm, beta_hbm,
            out_hbm, ids_v, bias_v, rows_v, ostage_v, ss_v, type_v, gamma_v,
            beta_v, gsem0, gsem1, psem0, psem1, osem):
        wid = lax.axis_index("s") * NC + lax.axis_index("c")
        tile_p0 = wid * PPT
        pltpu.sync_copy(ids_hbm.at[:, pl.ds(tile_p0, PPT)], ids_v)
        pltpu.sync_copy(type_hbm.at[pl.ds(0, 1)], type_v)
        pltpu.sync_copy(gamma_hbm, gamma_v)
        pltpu.sync_copy(beta_hbm, beta_v)

        gsems = (gsem0, gsem1)
        psems = (psem0, psem1)

        def issue(P, s):
            p0 = tile_p0 + s * PB
            pltpu.make_async_copy(
                pos_hbm.at[pl.ds(p0, PB)], bias_v.at[P], psems[P]).start()
            for b in range(B):
                idx = ids_v.at[b, pl.ds(s * PB, PB)]
                pltpu.make_async_copy(
                    word_hbm.at[idx], rows_v.at[P, b], gsems[P]).start()

        def wait_set(P, s):
            p0 = tile_p0 + s * PB
            pltpu.make_async_copy(
                pos_hbm.at[pl.ds(p0, PB)], bias_v.at[P], psems[P]).wait()
            for b in range(B):
                idx = ids_v.at[b, pl.ds(s * PB, PB)]
                pltpu.make_async_copy(
                    word_hbm.at[idx], rows_v.at[P, b], gsems[P]).wait()

        def drain_out(s):
            p0 = tile_p0 + s * PB
            for b in range(B):
                pltpu.make_async_copy(
                    ostage_v.at[b],
                    out_hbm.at[b, pl.ds(p0, PB)], osem).wait()

        def issue_out(s):
            p0 = tile_p0 + s * PB
            for b in range(B):
                pltpu.make_async_copy(
                    ostage_v.at[b],
                    out_hbm.at[b, pl.ds(p0, PB)], osem).start()

        def moment_pass(P):
            for p in range(PB):
                def add_type(jj, _, p=p):
                    for u in range(U):
                        sl = pl.ds(jj * (U * L) + u * L, L)
                        bias_v[P, p, sl] = bias_v[P, p, sl] + type_v[0, sl]
                    return 0
                lax.fori_loop(0, NCH // U, add_type, 0)

            def token(t, _):
                def stats(jj, carry):
                    acc = list(carry)
                    for u in range(U):
                        sl = pl.ds(jj * (U * L) + u * L, L)
                        bias_c = bias_v[P, t, sl]
                        for b in range(B):
                            x = rows_v[P, b, t, sl] + bias_c
                            acc[2 * b] = acc[2 * b] + x
                            acc[2 * b + 1] = acc[2 * b + 1] + x * x
                    return tuple(acc)

                zero = jnp.zeros((L,), jnp.float32)
                moments = lax.fori_loop(0, NCH // U, stats, (zero,) * (2 * B))
                for b in range(B):
                    mean = _lane_sum(moments[2 * b]) * jnp.float32(1.0 / H)
                    ex2 = _lane_sum(moments[2 * b + 1]) * jnp.float32(1.0 / H)
                    var = ex2 - mean * mean
                    rstd = _rsqrt(var + jnp.float32(EPS))
                    ss_v[0, b, t, :] = rstd
                    ss_v[1, b, t, :] = -mean * rstd
                return 0
            lax.fori_loop(0, PB, token, 0)

        def norm_pass(P):
            def token(t, _):
                scale = [ss_v[0, b, t, :] for b in range(B)]
                shift = [ss_v[1, b, t, :] for b in range(B)]

                def norm(jj, _):
                    for u in range(U):
                        sl = pl.ds(jj * (U * L) + u * L, L)
                        bias_c = bias_v[P, t, sl]
                        g = gamma_v[sl]
                        be = beta_v[sl]
                        for b in range(B):
                            x = rows_v[P, b, t, sl] + bias_c
                            y = x * scale[b] + shift[b]
                            ostage_v[b, t, sl] = y * g + be
                    return 0
                return lax.fori_loop(0, NCH // U, norm, 0)
            lax.fori_loop(0, PB, token, 0)

        issue(0, 0)
        issue(1, 1)

        def step(h, _):
            for P in range(2):
                s = 2 * h + P
                wait_set(P, s)
                moment_pass(P)

                @pl.when(s >= 1)
                def _():
                    drain_out(s - 1)

                norm_pass(P)
                issue_out(s)

                @pl.when(s + 2 < NSB)
                def _():
                    issue(P, s + 2)
            return 0
        lax.fori_loop(0, NSB // 2, step, 0)
        drain_out(NSB - 1)

    return run(input_ids, word_emb, pos_emb, type_emb, gamma, beta)


# speedup vs baseline: 5.1708x; 3.6313x over previous
import functools

import jax
import jax.numpy as jnp
from jax import lax
from jax.experimental import pallas as pl
from jax.experimental.pallas import tpu as pltpu
from jax.experimental.pallas import tpu_sc as plsc

B = 4
S = 4096
H = 1024
EPS = 1e-12
L = 16
NC, NS = 2, 16
NW = NC * NS
PPT = S // NW
PB = 8
NSB = PPT // PB
NCH = H // L
U = 8


def _rsqrt(x):
    i = lax.bitcast_convert_type(x, jnp.int32)
    i = jnp.int32(0x5F3759DF) - lax.shift_right_logical(i, 1)
    y = lax.bitcast_convert_type(i, jnp.float32)
    for _ in range(4):
        y = y * (jnp.float32(1.5) - jnp.float32(0.5) * x * y * y)
    return y


_GATHER_DNUMS = lax.GatherDimensionNumbers(
    offset_dims=(), collapsed_slice_dims=(0,), start_index_map=(0,))


def _shuffle(v, perm):
    return lax.gather(v, perm[:, None], _GATHER_DNUMS, slice_sizes=(1,),
                      mode=lax.GatherScatterMode.PROMISE_IN_BOUNDS)


def _lane_sum(v):
    iota = lax.iota(jnp.int32, L)
    for k in (8, 4, 2, 1):
        v = v + _shuffle(v, jnp.bitwise_xor(iota, jnp.int32(k)))
    return v


def kernel(input_ids, word_emb, pos_emb, type_emb, gamma, beta):
    mesh = plsc.VectorSubcoreMesh(core_axis_name="c", subcore_axis_name="s")

    @functools.partial(
        pl.kernel,
        mesh=mesh,
        out_type=jax.ShapeDtypeStruct((B, S, H), jnp.float32),
        scratch_types=[
            pltpu.VMEM((B, PPT), jnp.int32),
            pltpu.VMEM((2, PB, H), jnp.float32),
            pltpu.VMEM((2, B, PB, H), jnp.float32),
            pltpu.VMEM((B, PB, H), jnp.float32),
            pltpu.VMEM((2, B, PB, L), jnp.float32),
            pltpu.VMEM((1, H), jnp.float32),
            pltpu.VMEM((H,), jnp.float32),
            pltpu.VMEM((H,), jnp.float32),
            pltpu.SemaphoreType.DMA,
            pltpu.SemaphoreType.DMA,
            pltpu.SemaphoreType.DMA,
            pltpu.SemaphoreType.DMA,
            pltpu.SemaphoreType.DMA,
        ],
    )
    def run(ids_hbm, word_hbm, pos_hbm, type_hbm, gamma_hbm, beta_hbm,
            out_hbm, ids_v, bias_v, rows_v, ostage_v, ss_v, type_v, gamma_v,
            beta_v, gsem0, gsem1, psem0, psem1, osem):
        wid = lax.axis_index("s") * NC + lax.axis_index("c")
        tile_p0 = wid * PPT
        pltpu.sync_copy(ids_hbm.at[:, pl.ds(tile_p0, PPT)], ids_v)
        pltpu.sync_copy(type_hbm.at[pl.ds(0, 1)], type_v)
        pltpu.sync_copy(gamma_hbm, gamma_v)
        pltpu.sync_copy(beta_hbm, beta_v)

        gsems = (gsem0, gsem1)
        psems = (psem0, psem1)

        def issue(P, s):
            p0 = tile_p0 + s * PB
            pltpu.make_async_copy(
                pos_hbm.at[pl.ds(p0, PB)], bias_v.at[P], psems[P]).start()
            for b in range(B):
                idx = ids_v.at[b, pl.ds(s * PB, PB)]
                pltpu.make_async_copy(
                    word_hbm.at[idx], rows_v.at[P, b], gsems[P]).start()

        def wait_set(P, s):
            p0 = tile_p0 + s * PB
            pltpu.make_async_copy(
                pos_hbm.at[pl.ds(p0, PB)], bias_v.at[P], psems[P]).wait()
            for b in range(B):
                idx = ids_v.at[b, pl.ds(s * PB, PB)]
                pltpu.make_async_copy(
                    word_hbm.at[idx], rows_v.at[P, b], gsems[P]).wait()

        def drain_out(s):
            p0 = tile_p0 + s * PB
            for b in range(B):
                pltpu.make_async_copy(
                    ostage_v.at[b],
                    out_hbm.at[b, pl.ds(p0, PB)], osem).wait()

        def issue_out(s):
            p0 = tile_p0 + s * PB
            for b in range(B):
                pltpu.make_async_copy(
                    ostage_v.at[b],
                    out_hbm.at[b, pl.ds(p0, PB)], osem).start()

        def moment_pass(P):
            for p in range(PB):
                @plsc.parallel_loop(0, NCH, unroll=U)
                def _(j, p=p):
                    sl = pl.ds(j * L, L)
                    bias_v[P, p, sl] = bias_v[P, p, sl] + type_v[0, sl]

            def token(t, _):
                zero = jnp.zeros((L,), jnp.float32)

                @plsc.parallel_loop(0, NCH, unroll=U, carry=(zero,) * (2 * B))
                def moments(j, carry):
                    acc = list(carry)
                    sl = pl.ds(j * L, L)
                    bias_c = bias_v[P, t, sl]
                    for b in range(B):
                        x = rows_v[P, b, t, sl] + bias_c
                        acc[2 * b] = acc[2 * b] + x
                        acc[2 * b + 1] = acc[2 * b + 1] + x * x
                    return tuple(acc)

                for b in range(B):
                    mean = _lane_sum(moments[2 * b]) * jnp.float32(1.0 / H)
                    ex2 = _lane_sum(moments[2 * b + 1]) * jnp.float32(1.0 / H)
                    var = ex2 - mean * mean
                    rstd = _rsqrt(var + jnp.float32(EPS))
                    ss_v[0, b, t, :] = rstd
                    ss_v[1, b, t, :] = -mean * rstd
                return 0
            lax.fori_loop(0, PB, token, 0)

        def norm_pass(P):
            def token(t, _):
                scale = [ss_v[0, b, t, :] for b in range(B)]
                shift = [ss_v[1, b, t, :] for b in range(B)]

                @plsc.parallel_loop(0, NCH, unroll=U)
                def _(j):
                    sl = pl.ds(j * L, L)
                    bias_c = bias_v[P, t, sl]
                    g = gamma_v[sl]
                    be = beta_v[sl]
                    for b in range(B):
                        x = rows_v[P, b, t, sl] + bias_c
                        y = x * scale[b] + shift[b]
                        ostage_v[b, t, sl] = y * g + be
                return 0
            lax.fori_loop(0, PB, token, 0)

        issue(0, 0)
        issue(1, 1)

        def step(h, _):
            for P in range(2):
                s = 2 * h + P
                wait_set(P, s)
                moment_pass(P)

                @pl.when(s >= 1)
                def _():
                    drain_out(s - 1)

                norm_pass(P)
                issue_out(s)

                @pl.when(s + 2 < NSB)
                def _():
                    issue(P, s + 2)
            return 0
        lax.fori_loop(0, NSB // 2, step, 0)
        drain_out(NSB - 1)

    return run(input_ids, word_emb, pos_emb, type_emb, gamma, beta)
